# baseline (speedup 1.0000x reference)
import functools

import jax
import jax.numpy as jnp
from jax import lax
from jax.experimental import pallas as pl
from jax.experimental.pallas import tpu as pltpu
from jax.experimental.pallas import tpu_sc as plsc

B = 16384
D = 64
EPS = 1e-3

NC = 2
NS = 16
NW = NC * NS
BPW = B // NW
CHUNK = 128
NCHUNK = BPW // CHUNK


def _sc_gather(user_ids, joke_ids, user_table, joke_table):
    mesh = plsc.VectorSubcoreMesh(core_axis_name="c", subcore_axis_name="s")

    @functools.partial(
        pl.kernel,
        out_type=(
            jax.ShapeDtypeStruct((B, D), jnp.float32),
            jax.ShapeDtypeStruct((B, D), jnp.float32),
        ),
        mesh=mesh,
        scratch_types=[
            pltpu.VMEM((NCHUNK, CHUNK), jnp.int32),
            pltpu.VMEM((NCHUNK, CHUNK), jnp.int32),
            pltpu.VMEM((BPW, D), jnp.float32),
            pltpu.VMEM((BPW, D), jnp.float32),
            pltpu.SemaphoreType.DMA,
        ],
        compiler_params=pltpu.CompilerParams(use_tc_tiling_on_sc=False),
    )
    def gather_kernel(uid_hbm, jid_hbm, utab_hbm, jtab_hbm,
                      uout_hbm, jout_hbm,
                      uidx_v, jidx_v, urows_v, jrows_v, sem):
        wid = lax.axis_index("s") * NC + lax.axis_index("c")
        base = wid * BPW
        for j in range(NCHUNK):
            pltpu.sync_copy(uid_hbm.at[pl.ds(base + j * CHUNK, CHUNK)],
                            uidx_v.at[j])
            pltpu.sync_copy(jid_hbm.at[pl.ds(base + j * CHUNK, CHUNK)],
                            jidx_v.at[j])
        copies = []
        for j in range(NCHUNK):
            copies.append(pltpu.async_copy(
                utab_hbm.at[uidx_v.at[j]],
                urows_v.at[pl.ds(j * CHUNK, CHUNK)], sem))
            copies.append(pltpu.async_copy(
                jtab_hbm.at[jidx_v.at[j]],
                jrows_v.at[pl.ds(j * CHUNK, CHUNK)], sem))
        for c in copies:
            c.wait()
        pltpu.sync_copy(urows_v, uout_hbm.at[pl.ds(base, BPW)])
        pltpu.sync_copy(jrows_v, jout_hbm.at[pl.ds(base, BPW)])

    return gather_kernel(user_ids, joke_ids, user_table, joke_table)


def _mlp_body(u_ref, j_ref, w1_ref, b1_ref, w2_ref, b2_ref, w3_ref, b3_ref,
              w4_ref, b4_ref, w5_ref, b5_ref, out_ref):
    x = jnp.concatenate([u_ref[...], j_ref[...]], axis=1)
    h = jnp.maximum(jnp.dot(x, w1_ref[...],
                            preferred_element_type=jnp.float32,
                            precision=jax.lax.Precision.HIGHEST) + b1_ref[...], 0.0)
    h = jnp.maximum(jnp.dot(h, w2_ref[...],
                            preferred_element_type=jnp.float32,
                            precision=jax.lax.Precision.HIGHEST) + b2_ref[...], 0.0)
    h = jnp.maximum(jnp.dot(h, w3_ref[...],
                            preferred_element_type=jnp.float32,
                            precision=jax.lax.Precision.HIGHEST) + b3_ref[...], 0.0)
    h = jnp.maximum(jnp.dot(h, w4_ref[...],
                            preferred_element_type=jnp.float32,
                            precision=jax.lax.Precision.HIGHEST) + b4_ref[...], 0.0)
    h = jnp.maximum(jnp.dot(h, w5_ref[...],
                            preferred_element_type=jnp.float32,
                            precision=jax.lax.Precision.HIGHEST) + b5_ref[...], 0.0)
    out_ref[...] = h


def _tc_mlp(user, joke, W1, b1, W2, b2, W3, b3, W4, b4, W5, b5):
    blk = 2048
    grid = (B // blk,)
    full = lambda a: pl.BlockSpec(a.shape, lambda i: (0,) * a.ndim)
    return pl.pallas_call(
        _mlp_body,
        grid=grid,
        in_specs=[
            pl.BlockSpec((blk, D), lambda i: (i, 0)),
            pl.BlockSpec((blk, D), lambda i: (i, 0)),
            full(W1), full(b1), full(W2), full(b2), full(W3), full(b3),
            full(W4), full(b4), full(W5), full(b5),
        ],
        out_specs=pl.BlockSpec((blk, 1), lambda i: (i, 0)),
        out_shape=jax.ShapeDtypeStruct((B, 1), jnp.float32),
    )(user, joke, W1, b1, W2, b2, W3, b3, W4, b4, W5, b5)


def kernel(user_ids, joke_ids, user_table, joke_table,
           W1, b1, g1, be1, W2, b2, g2, be2, W3, b3, W4, b4, W5, b5):
    user, joke = _sc_gather(user_ids.astype(jnp.int32),
                            joke_ids.astype(jnp.int32),
                            user_table, joke_table)
    inv = 1.0 / jnp.sqrt(jnp.float32(1.0 + EPS))
    s1 = g1 * inv
    W2f = s1[:, None] * W2
    b2f = be1 @ W2 + b2
    s2 = g2 * inv
    W3f = s2[:, None] * W3
    b3f = be2 @ W3 + b3
    return _tc_mlp(user, joke,
                   W1, b1[None, :], W2f, b2f[None, :], W3f, b3f[None, :],
                   W4, b4[None, :], W5, b5[None, :])
